# baseline (speedup 1.0000x reference)
import functools

import jax
import jax.numpy as jnp
import numpy as np
from jax import lax
from jax.experimental import pallas as pl
from jax.experimental.pallas import tpu as pltpu
from jax.experimental.pallas import tpu_sc as plsc

NUM_CLASSES = 19
IGNORE = 255
DIM = 256
REGION_MEM = 2000
PIXEL_MEM = 20000
PIXEL_CONTRAST = 4096 // NUM_CLASSES + 1
REGION_CONTRAST = 1024 // NUM_CLASSES + 1
TAU_C = 0.1
KD_T = 1.0
MAX_SAMPLES = 1024
PIX_UPD = 10
LW_PIX = 0.1
LW_REG = 0.1

M = 8192
CHUNK = 512
NCHUNK = M // CHUNK
KPAD = 192
NPIX = NUM_CLASSES * PIXEL_CONTRAST
NREG = NUM_CLASSES * REGION_CONTRAST
NW = 32
NPIX_SC = 4352
NREG_SC = 1280
PB = NPIX_SC // NW
RB = NREG_SC // NW

_PIDX = np.array([
    19851, 12832, 2748, 10523, 1960, 5101, 10204, 14383, 8490, 8589, 7203,
    13428, 2994, 7745, 16530, 9747, 15513, 10494, 11667, 1697, 16122, 17138,
    15651, 19828, 8375, 10461, 6872, 18476, 9449, 10646, 8416, 797, 11263,
    2182, 9573, 10059, 15041, 6983, 3116, 18154, 3046, 12007, 8180, 13800,
    14128, 3207, 18959, 12575, 5344, 12351, 15909, 2261, 13268, 13183, 18122,
    2529, 4684, 10331, 11933, 4549, 8970, 8549, 13137, 15150, 15675, 13074,
    19287, 3038, 4685, 14202, 32, 15331, 13996, 19724, 8289, 14748, 3146,
    11400, 8388, 12080, 16497, 886, 5079, 5271, 1386, 6805, 18926, 6182,
    18284, 14273, 17271, 4667, 13937, 17759, 10745, 8206, 1692, 11015, 3746,
    13444, 2580, 2734, 4544, 5468, 12671, 4416, 16991, 11227, 19270, 5295,
    11974, 6850, 9245, 6058, 16590, 14973, 5521, 3692, 3623, 4204, 4224,
    17054, 4744, 15849, 8733, 10963, 2489, 14426, 4747, 17117, 11126, 17410,
    15315, 7495, 3616, 8960, 9836, 1280, 1597, 2322, 15244, 2129, 6593,
    16353, 18690, 8726, 6863, 6085, 17385, 10050, 14322, 10388, 206, 3778,
    11961, 4109, 10799, 9723, 19031, 9039, 19086, 14720, 11385, 12325, 1564,
    1471, 7612, 4989, 4659, 19561, 1843, 9986, 15303, 16629, 6853, 15096,
    15294, 4438, 19374, 1226, 11689, 9025, 16624, 4897, 14948, 13578, 14308,
    17701, 9489, 543, 3926, 9700, 16286, 7649, 19236, 13304, 6473, 13249,
    10943, 6016, 14963, 408, 19324, 16118, 15221, 483, 4915, 12933, 16443,
    2306, 16188, 4682, 18063, 16821, 7018, 5746], dtype=np.int32)

_RIDX = np.array([
    1858, 1255, 1078, 297, 1329, 1302, 1072, 900, 1014, 185, 1354, 1985,
    1053, 678, 1348, 454, 1309, 1361, 1668, 664, 1450, 1031, 15, 318, 859,
    1525, 1146, 89, 253, 606, 1318, 115, 1898, 686, 839, 258, 586, 1826,
    1079, 1474, 1911, 1857, 437, 1831, 1803, 1912, 452, 713, 1083, 892, 1086,
    879, 1446, 1147], dtype=np.int32)


def _dg(a, b, ca, cb):
    return lax.dot_general(a, b, (((ca,), (cb,)), ((), ())),
                           preferred_element_type=jnp.float32)


def _prep_body(lab_ref, s_ref, t_ref, w1_ref, g_ref, b_ref, w2_ref,
               sa_ref, ta_ref, mf_ref, upd_ref, cnt_ref,
               xa, tas, ssum, ssq, segs, updacc, basec, cntrow):
    i = pl.program_id(0)

    @pl.when(i == 0)
    def _init():
        ssum[...] = jnp.zeros_like(ssum)
        ssq[...] = jnp.zeros_like(ssq)
        segs[...] = jnp.zeros_like(segs)
        updacc[...] = jnp.zeros_like(updacc)
        basec[...] = jnp.zeros_like(basec)
        cntrow[...] = jnp.zeros_like(cntrow)

    lab = lab_ref[0]
    tb = t_ref[...].reshape(DIM, CHUNK)
    tn = tb / (jnp.sqrt(jnp.sum(tb * tb, axis=0, keepdims=True)) + 1e-12)
    cls = lax.broadcasted_iota(jnp.int32, (NUM_CLASSES, CHUNK), 0)
    oh = jnp.where((lab == cls) & (lab != IGNORE), 1.0, 0.0)

    cnt_b = jnp.sum(oh, axis=1, keepdims=True)
    ri = lax.broadcasted_iota(jnp.int32, (CHUNK, CHUNK), 0)
    ci = lax.broadcasted_iota(jnp.int32, (CHUNK, CHUNK), 1)
    sup = jnp.where(ri < ci, 1.0, 0.0)
    excl = _dg(oh, sup, 1, 0) + basec[...]
    basec[...] = basec[...] + cnt_b
    segs[...] = segs[...] + _dg(tn, oh, 1, 1)
    cntrow[...] = cntrow[...] + _dg(jnp.ones((1, CHUNK), jnp.float32), oh, 1, 1)

    ecls = lax.broadcasted_iota(jnp.int32, (NUM_CLASSES, KPAD), 0)
    ecol = lax.broadcasted_iota(jnp.int32, (NUM_CLASSES, KPAD), 1)
    emat = jnp.where(ecol // PIX_UPD == ecls, 1.0, 0.0)
    clsw = _dg(emat, oh, 0, 0)
    exw = _dg(emat, excl, 0, 0)
    kcol = (lax.broadcasted_iota(jnp.int32, (KPAD, 1), 0) % PIX_UPD
            ).astype(jnp.float32)
    sel = clsw * jnp.where(exw == kcol, 1.0, 0.0)
    updacc[...] = updacc[...] + _dg(tn, sel, 1, 1)

    x1 = _dg(w1_ref[...], s_ref[...].reshape(512, CHUNK), 1, 0)
    ssum[...] = ssum[...] + jnp.sum(x1, axis=1, keepdims=True)
    ssq[...] = ssq[...] + jnp.sum(x1 * x1, axis=1, keepdims=True)

    @pl.when(i < MAX_SAMPLES // CHUNK)
    def _store():
        xa[:, pl.ds(i * CHUNK, CHUNK)] = x1
        tas[:, pl.ds(i * CHUNK, CHUNK)] = tn

    @pl.when(i == NCHUNK - 1)
    def _final():
        mean = ssum[...] / float(M)
        var = ssq[...] / float(M) - mean * mean
        xn = (xa[...] - mean) / jnp.sqrt(var + 1e-5) * g_ref[...] + b_ref[...]
        xn = jnp.maximum(xn, 0.0)
        s2 = _dg(w2_ref[...], xn, 1, 0)
        sa_ref[...] = s2 / (jnp.sqrt(jnp.sum(s2 * s2, axis=0, keepdims=True))
                            + 1e-12)
        ta_ref[...] = tas[...]
        mf = segs[...] / jnp.maximum(cntrow[...], 1.0)
        mf_ref[...] = mf / (jnp.sqrt(jnp.sum(mf * mf, axis=0, keepdims=True))
                            + 1e-12)
        u = updacc[...]
        upd_ref[...] = u / (jnp.sqrt(jnp.sum(u * u, axis=0, keepdims=True))
                            + 1e-12)
        cnt_ref[...] = basec[...]


def _prep(lab3, s3, t3, W1, gamma, beta, W2):
    f32 = jnp.float32
    return pl.pallas_call(
        _prep_body,
        grid=(NCHUNK,),
        in_specs=[
            pl.BlockSpec((1, 1, CHUNK), lambda i: (i, 0, 0)),
            pl.BlockSpec((1, 512, 8, 64), lambda i: (i // 8, 0, i % 8, 0)),
            pl.BlockSpec((1, DIM, 8, 64), lambda i: (i // 8, 0, i % 8, 0)),
            pl.BlockSpec((DIM, 512), lambda i: (0, 0)),
            pl.BlockSpec((DIM, 1), lambda i: (0, 0)),
            pl.BlockSpec((DIM, 1), lambda i: (0, 0)),
            pl.BlockSpec((DIM, DIM), lambda i: (0, 0)),
        ],
        out_specs=[
            pl.BlockSpec((DIM, MAX_SAMPLES), lambda i: (0, 0)),
            pl.BlockSpec((DIM, MAX_SAMPLES), lambda i: (0, 0)),
            pl.BlockSpec((DIM, NUM_CLASSES), lambda i: (0, 0)),
            pl.BlockSpec((DIM, KPAD), lambda i: (0, 0)),
            pl.BlockSpec((NUM_CLASSES, 1), lambda i: (0, 0)),
        ],
        out_shape=[
            jax.ShapeDtypeStruct((DIM, MAX_SAMPLES), f32),
            jax.ShapeDtypeStruct((DIM, MAX_SAMPLES), f32),
            jax.ShapeDtypeStruct((DIM, NUM_CLASSES), f32),
            jax.ShapeDtypeStruct((DIM, KPAD), f32),
            jax.ShapeDtypeStruct((NUM_CLASSES, 1), f32),
        ],
        scratch_shapes=[
            pltpu.VMEM((DIM, MAX_SAMPLES), f32),
            pltpu.VMEM((DIM, MAX_SAMPLES), f32),
            pltpu.VMEM((DIM, 1), f32),
            pltpu.VMEM((DIM, 1), f32),
            pltpu.VMEM((DIM, NUM_CLASSES), f32),
            pltpu.VMEM((DIM, KPAD), f32),
            pltpu.VMEM((NUM_CLASSES, 1), f32),
            pltpu.VMEM((1, NUM_CLASSES), f32),
        ],
    )(lab3, s3, t3, W1, gamma, beta, W2)


def _sc_gather(ptab, pidx_pad, rtab, ridx_pad):
    mesh = plsc.VectorSubcoreMesh(core_axis_name="c", subcore_axis_name="s")

    @functools.partial(
        pl.kernel, mesh=mesh,
        out_type=[jax.ShapeDtypeStruct((NPIX_SC, DIM), jnp.float32),
                  jax.ShapeDtypeStruct((NREG_SC, DIM), jnp.float32)],
        scratch_types=[
            pltpu.VMEM((PB,), jnp.int32),
            pltpu.VMEM((PB, DIM), jnp.float32),
            pltpu.VMEM((RB,), jnp.int32),
            pltpu.VMEM((RB, DIM), jnp.float32),
            pltpu.SemaphoreType.DMA,
            pltpu.SemaphoreType.DMA,
        ],
    )
    def gk(ptab_h, pidx_h, rtab_h, ridx_h, outp_h, outr_h,
           pidx_v, prow_v, ridx_v, rrow_v, sem1, sem2):
        wid = lax.axis_index("s") * 2 + lax.axis_index("c")
        pb = wid * PB
        rb = wid * RB
        pltpu.sync_copy(pidx_h.at[pl.ds(pb, PB)], pidx_v)
        cp1 = pltpu.async_copy(ptab_h.at[pidx_v], prow_v, sem1)
        pltpu.sync_copy(ridx_h.at[pl.ds(rb, RB)], ridx_v)
        cp2 = pltpu.async_copy(rtab_h.at[ridx_v], rrow_v, sem2)
        cp1.wait()
        cp2.wait()
        pltpu.sync_copy(prow_v, outp_h.at[pl.ds(pb, PB)])
        pltpu.sync_copy(rrow_v, outr_h.at[pl.ds(rb, RB)])

    return gk(ptab, pidx_pad, rtab, ridx_pad)


def _loss_body(sa_ref, ta_ref, xpb_ref, xrb_ref, pidx_ref, ridx_ref,
               pptr_ref, rptr_ref, cnt_ref, upd_ref, mf_ref, lab_ref,
               lp_ref, lr_ref, xps, xrs, accs):
    i = pl.program_id(0)

    @pl.when(i == 0)
    def _init():
        rowp = lax.broadcasted_iota(jnp.int32, (NPIX_SC, 1), 0)
        cp = jnp.minimum(rowp // PIXEL_CONTRAST, NUM_CLASSES - 1)
        ohc = jnp.where(
            cp == lax.broadcasted_iota(jnp.int32, (NPIX_SC, NUM_CLASSES), 1),
            1.0, 0.0)
        ptr_e = _dg(ohc, pptr_ref[...], 1, 0)
        cnt_e = _dg(ohc, cnt_ref[...], 1, 0)
        kk = jnp.mod(pidx_ref[...] - ptr_e, float(PIXEL_MEM))
        mp = jnp.where(
            (kk < PIX_UPD) & (kk < cnt_e) & (rowp < NPIX), 1.0, 0.0)
        srci = (cp * PIX_UPD
                + jnp.minimum(kk, PIX_UPD - 1).astype(jnp.int32))
        ohp = jnp.where(
            srci == lax.broadcasted_iota(jnp.int32, (NPIX_SC, KPAD), 1),
            1.0, 0.0)
        rp = _dg(ohp, upd_ref[...], 1, 1)
        xps[...] = xpb_ref[...] + mp * (rp - xpb_ref[...])

        rowr = lax.broadcasted_iota(jnp.int32, (NREG_SC, 1), 0)
        cr = jnp.minimum(rowr // REGION_CONTRAST, NUM_CLASSES - 1)
        ohcr = jnp.where(
            cr == lax.broadcasted_iota(jnp.int32, (NREG_SC, NUM_CLASSES), 1),
            1.0, 0.0)
        rptr_e = _dg(ohcr, rptr_ref[...], 1, 0)
        rcnt_e = _dg(ohcr, cnt_ref[...], 1, 0)
        mr = jnp.where(
            (ridx_ref[...] == rptr_e) & (rcnt_e > 0) & (rowr < NREG),
            1.0, 0.0)
        rr = _dg(ohcr, mf_ref[...], 1, 1)
        xrs[...] = xrb_ref[...] + mr * (rr - xrb_ref[...])
        accs[0] = 0.0
        accs[1] = 0.0
        accs[2] = 0.0

    sa = sa_ref[...]
    ta = ta_ref[...]
    wa = jnp.where(lab_ref[...] != IGNORE, 1.0, 0.0)

    def kd_part(x, nvalid, ncols):
        msk = lax.broadcasted_iota(jnp.int32, (1, ncols), 1) < nvalid
        zs = _dg(sa, x, 0, 1) * (1.0 / TAU_C / KD_T)
        zt = _dg(ta, x, 0, 1) * (1.0 / TAU_C / KD_T)
        zs = jnp.where(msk, zs, -12000.0)
        zt = jnp.where(msk, zt, -12000.0)
        ezs = jnp.exp(zs)
        ezt = jnp.exp(zt)
        ss = jnp.sum(ezs, axis=1, keepdims=True)
        st = jnp.sum(ezt, axis=1, keepdims=True)
        logps = zs - jnp.log(ss)
        logpt = zt - jnp.log(st)
        pt = ezt / st
        kl = jnp.sum(pt * (logpt - logps), axis=1, keepdims=True)
        return jnp.sum(kl * wa)

    vp = kd_part(xps[...], NPIX, NPIX_SC)
    vr = kd_part(xrs[...], NREG, NREG_SC)
    accs[0] = accs[0] + vp
    accs[1] = accs[1] + vr
    accs[2] = accs[2] + jnp.sum(wa)

    @pl.when(i == pl.num_programs(0) - 1)
    def _fin():
        den = jnp.maximum(accs[2], 1.0)
        lp_ref[...] = jnp.broadcast_to(
            accs[0] / den * (KD_T * KD_T) * LW_PIX, (1, 1))
        lr_ref[...] = jnp.broadcast_to(
            accs[1] / den * (KD_T * KD_T) * LW_REG, (1, 1))


def _loss(sa, ta, xpb, xrb, pidxf, ridxf, pptrf, rptrf, cnt, upd, mf, laba):
    f32 = jnp.float32
    blk = 128
    return pl.pallas_call(
        _loss_body,
        grid=(MAX_SAMPLES // blk,),
        in_specs=[
            pl.BlockSpec((DIM, blk), lambda i: (0, i)),
            pl.BlockSpec((DIM, blk), lambda i: (0, i)),
            pl.BlockSpec((NPIX_SC, DIM), lambda i: (0, 0)),
            pl.BlockSpec((NREG_SC, DIM), lambda i: (0, 0)),
            pl.BlockSpec((NPIX_SC, 1), lambda i: (0, 0)),
            pl.BlockSpec((NREG_SC, 1), lambda i: (0, 0)),
            pl.BlockSpec((NUM_CLASSES, 1), lambda i: (0, 0)),
            pl.BlockSpec((NUM_CLASSES, 1), lambda i: (0, 0)),
            pl.BlockSpec((NUM_CLASSES, 1), lambda i: (0, 0)),
            pl.BlockSpec((DIM, KPAD), lambda i: (0, 0)),
            pl.BlockSpec((DIM, NUM_CLASSES), lambda i: (0, 0)),
            pl.BlockSpec((blk, 1), lambda i: (i, 0)),
        ],
        out_specs=[
            pl.BlockSpec((1, 1), lambda i: (0, 0)),
            pl.BlockSpec((1, 1), lambda i: (0, 0)),
        ],
        out_shape=[
            jax.ShapeDtypeStruct((1, 1), f32),
            jax.ShapeDtypeStruct((1, 1), f32),
        ],
        scratch_shapes=[
            pltpu.VMEM((NPIX_SC, DIM), f32),
            pltpu.VMEM((NREG_SC, DIM), f32),
            pltpu.SMEM((4,), f32),
        ],
    )(sa, ta, xpb, xrb, pidxf, ridxf, pptrf, rptrf, cnt, upd, mf, laba)


def kernel(s_feats, t_feats, logits_S, logits_T, labels, W1, gamma, beta, W2,
           seg_queue, pix_queue, seg_ptr, pix_ptr):
    f32 = jnp.float32
    lab = labels[:, 0, ::8, ::8].reshape(M)
    lab3 = lab.reshape(NCHUNK, 1, CHUNK)

    sa, ta, mean_feat, upd, cnt_col = _prep(
        lab3, s_feats, t_feats, W1, gamma.reshape(DIM, 1),
        beta.reshape(DIM, 1), W2)

    pflat = np.concatenate(
        [(np.arange(NUM_CLASSES, dtype=np.int32)[:, None] * PIXEL_MEM
          + _PIDX[None, :]).reshape(NPIX),
         np.zeros(NPIX_SC - NPIX, np.int32)])
    rflat = np.concatenate(
        [(np.arange(NUM_CLASSES, dtype=np.int32)[:, None] * REGION_MEM
          + _RIDX[None, :]).reshape(NREG),
         np.zeros(NREG_SC - NREG, np.int32)])
    xpb, xrb = _sc_gather(
        pix_queue.reshape(NUM_CLASSES * PIXEL_MEM, DIM), jnp.asarray(pflat),
        seg_queue.reshape(NUM_CLASSES * REGION_MEM, DIM), jnp.asarray(rflat))

    pidxf = jnp.asarray(
        np.concatenate([np.tile(_PIDX, NUM_CLASSES),
                        np.zeros(NPIX_SC - NPIX, np.int32)])
        .astype(np.float32).reshape(NPIX_SC, 1))
    ridxf = jnp.asarray(
        np.concatenate([np.tile(_RIDX, NUM_CLASSES),
                        np.full(NREG_SC - NREG, -1, np.int32)])
        .astype(np.float32).reshape(NREG_SC, 1))
    pptrf = pix_ptr.astype(f32).reshape(NUM_CLASSES, 1)
    rptrf = seg_ptr.astype(f32).reshape(NUM_CLASSES, 1)
    laba = lab[:MAX_SAMPLES].reshape(MAX_SAMPLES, 1)

    lp_out, lr_out = _loss(sa, ta, xpb, xrb, pidxf, ridxf, pptrf, rptrf,
                           cnt_col, upd, mean_feat, laba)
    return (lp_out[0, 0], lr_out[0, 0])


# speedup vs baseline: 1.5446x; 1.5446x over previous
import functools

import jax
import jax.numpy as jnp
import numpy as np
from jax import lax
from jax.experimental import pallas as pl
from jax.experimental.pallas import tpu as pltpu
from jax.experimental.pallas import tpu_sc as plsc

NUM_CLASSES = 19
IGNORE = 255
DIM = 256
REGION_MEM = 2000
PIXEL_MEM = 20000
PIXEL_CONTRAST = 4096 // NUM_CLASSES + 1
REGION_CONTRAST = 1024 // NUM_CLASSES + 1
TAU_C = 0.1
KD_T = 1.0
MAX_SAMPLES = 1024
PIX_UPD = 10
LW_PIX = 0.1
LW_REG = 0.1

M = 8192
CHUNK = 512
NCHUNK = M // CHUNK
KPAD = 192
NPIX = NUM_CLASSES * PIXEL_CONTRAST
NREG = NUM_CLASSES * REGION_CONTRAST
NW = 32
NPIX_SC = 4352
NREG_SC = 1280
PB = NPIX_SC // NW
RB = NREG_SC // NW

_PIDX = np.array([
    19851, 12832, 2748, 10523, 1960, 5101, 10204, 14383, 8490, 8589, 7203,
    13428, 2994, 7745, 16530, 9747, 15513, 10494, 11667, 1697, 16122, 17138,
    15651, 19828, 8375, 10461, 6872, 18476, 9449, 10646, 8416, 797, 11263,
    2182, 9573, 10059, 15041, 6983, 3116, 18154, 3046, 12007, 8180, 13800,
    14128, 3207, 18959, 12575, 5344, 12351, 15909, 2261, 13268, 13183, 18122,
    2529, 4684, 10331, 11933, 4549, 8970, 8549, 13137, 15150, 15675, 13074,
    19287, 3038, 4685, 14202, 32, 15331, 13996, 19724, 8289, 14748, 3146,
    11400, 8388, 12080, 16497, 886, 5079, 5271, 1386, 6805, 18926, 6182,
    18284, 14273, 17271, 4667, 13937, 17759, 10745, 8206, 1692, 11015, 3746,
    13444, 2580, 2734, 4544, 5468, 12671, 4416, 16991, 11227, 19270, 5295,
    11974, 6850, 9245, 6058, 16590, 14973, 5521, 3692, 3623, 4204, 4224,
    17054, 4744, 15849, 8733, 10963, 2489, 14426, 4747, 17117, 11126, 17410,
    15315, 7495, 3616, 8960, 9836, 1280, 1597, 2322, 15244, 2129, 6593,
    16353, 18690, 8726, 6863, 6085, 17385, 10050, 14322, 10388, 206, 3778,
    11961, 4109, 10799, 9723, 19031, 9039, 19086, 14720, 11385, 12325, 1564,
    1471, 7612, 4989, 4659, 19561, 1843, 9986, 15303, 16629, 6853, 15096,
    15294, 4438, 19374, 1226, 11689, 9025, 16624, 4897, 14948, 13578, 14308,
    17701, 9489, 543, 3926, 9700, 16286, 7649, 19236, 13304, 6473, 13249,
    10943, 6016, 14963, 408, 19324, 16118, 15221, 483, 4915, 12933, 16443,
    2306, 16188, 4682, 18063, 16821, 7018, 5746], dtype=np.int32)

_RIDX = np.array([
    1858, 1255, 1078, 297, 1329, 1302, 1072, 900, 1014, 185, 1354, 1985,
    1053, 678, 1348, 454, 1309, 1361, 1668, 664, 1450, 1031, 15, 318, 859,
    1525, 1146, 89, 253, 606, 1318, 115, 1898, 686, 839, 258, 586, 1826,
    1079, 1474, 1911, 1857, 437, 1831, 1803, 1912, 452, 713, 1083, 892, 1086,
    879, 1446, 1147], dtype=np.int32)


def _dg(a, b, ca, cb):
    return lax.dot_general(a, b, (((ca,), (cb,)), ((), ())),
                           preferred_element_type=jnp.float32)


def _prep_body(lab_ref, s_ref, t_ref, w1_ref, g_ref, b_ref, w2_ref,
               sa_ref, ta_ref, mf_ref, upd_ref, cnt_ref,
               xa, tas, ssum, ssq, segs, updacc, basec, cntcol):
    i = pl.program_id(0)

    @pl.when(i == 0)
    def _init():
        ssum[...] = jnp.zeros_like(ssum)
        ssq[...] = jnp.zeros_like(ssq)
        segs[...] = jnp.zeros_like(segs)
        updacc[...] = jnp.zeros_like(updacc)
        basec[...] = jnp.zeros_like(basec)
        cntcol[...] = jnp.zeros_like(cntcol)

    lab = lab_ref[...]
    tb = t_ref[...]
    tn = tb / (jnp.sqrt(jnp.sum(tb * tb, axis=1, keepdims=True)) + 1e-12)
    cls = lax.broadcasted_iota(jnp.int32, (CHUNK, NUM_CLASSES), 1)
    oh = jnp.where((lab == cls) & (lab != IGNORE), 1.0, 0.0)

    cnt_b = jnp.sum(oh, axis=0, keepdims=True)
    ri = lax.broadcasted_iota(jnp.int32, (CHUNK, CHUNK), 0)
    ci = lax.broadcasted_iota(jnp.int32, (CHUNK, CHUNK), 1)
    tril = jnp.where(ci < ri, 1.0, 0.0)
    excl = _dg(tril, oh, 1, 0) + basec[...]
    basec[...] = basec[...] + cnt_b
    segs[...] = segs[...] + _dg(oh, tn, 0, 0)
    cntcol[...] = cntcol[...] + _dg(oh, jnp.ones((CHUNK, 1), jnp.float32),
                                    0, 0)

    ecls = lax.broadcasted_iota(jnp.int32, (NUM_CLASSES, KPAD), 0)
    ecol = lax.broadcasted_iota(jnp.int32, (NUM_CLASSES, KPAD), 1)
    emat = jnp.where(ecol // PIX_UPD == ecls, 1.0, 0.0)
    clsw = _dg(oh, emat, 1, 0)
    exw = _dg(excl, emat, 1, 0)
    kvec = (lax.broadcasted_iota(jnp.int32, (1, KPAD), 1) % PIX_UPD
            ).astype(jnp.float32)
    sel = clsw * jnp.where(exw == kvec, 1.0, 0.0)
    updacc[...] = updacc[...] + _dg(sel, tn, 0, 0)

    x1 = _dg(s_ref[...], w1_ref[...], 1, 1)
    ssum[...] = ssum[...] + jnp.sum(x1, axis=0, keepdims=True)
    ssq[...] = ssq[...] + jnp.sum(x1 * x1, axis=0, keepdims=True)

    @pl.when(i < MAX_SAMPLES // CHUNK)
    def _store():
        xa[pl.ds(i * CHUNK, CHUNK), :] = x1
        tas[pl.ds(i * CHUNK, CHUNK), :] = tn

    @pl.when(i == NCHUNK - 1)
    def _final():
        mean = ssum[...] / float(M)
        var = ssq[...] / float(M) - mean * mean
        xn = (xa[...] - mean) / jnp.sqrt(var + 1e-5) * g_ref[...] + b_ref[...]
        xn = jnp.maximum(xn, 0.0)
        s2 = _dg(xn, w2_ref[...], 1, 1)
        sa_ref[...] = s2 / (jnp.sqrt(jnp.sum(s2 * s2, axis=1, keepdims=True))
                            + 1e-12)
        ta_ref[...] = tas[...]
        mf = segs[...] / jnp.maximum(cntcol[...], 1.0)
        mf_ref[...] = mf / (jnp.sqrt(jnp.sum(mf * mf, axis=1, keepdims=True))
                            + 1e-12)
        u = updacc[...]
        upd_ref[...] = u / (jnp.sqrt(jnp.sum(u * u, axis=1, keepdims=True))
                            + 1e-12)
        cnt_ref[...] = cntcol[...]


def _prep(lab2, sp, tp, W1, gamma, beta, W2):
    f32 = jnp.float32
    return pl.pallas_call(
        _prep_body,
        grid=(NCHUNK,),
        in_specs=[
            pl.BlockSpec((CHUNK, 1), lambda i: (i, 0)),
            pl.BlockSpec((CHUNK, 512), lambda i: (i, 0)),
            pl.BlockSpec((CHUNK, DIM), lambda i: (i, 0)),
            pl.BlockSpec((DIM, 512), lambda i: (0, 0)),
            pl.BlockSpec((1, DIM), lambda i: (0, 0)),
            pl.BlockSpec((1, DIM), lambda i: (0, 0)),
            pl.BlockSpec((DIM, DIM), lambda i: (0, 0)),
        ],
        out_specs=[
            pl.BlockSpec((MAX_SAMPLES, DIM), lambda i: (0, 0)),
            pl.BlockSpec((MAX_SAMPLES, DIM), lambda i: (0, 0)),
            pl.BlockSpec((NUM_CLASSES, DIM), lambda i: (0, 0)),
            pl.BlockSpec((KPAD, DIM), lambda i: (0, 0)),
            pl.BlockSpec((NUM_CLASSES, 1), lambda i: (0, 0)),
        ],
        out_shape=[
            jax.ShapeDtypeStruct((MAX_SAMPLES, DIM), f32),
            jax.ShapeDtypeStruct((MAX_SAMPLES, DIM), f32),
            jax.ShapeDtypeStruct((NUM_CLASSES, DIM), f32),
            jax.ShapeDtypeStruct((KPAD, DIM), f32),
            jax.ShapeDtypeStruct((NUM_CLASSES, 1), f32),
        ],
        scratch_shapes=[
            pltpu.VMEM((MAX_SAMPLES, DIM), f32),
            pltpu.VMEM((MAX_SAMPLES, DIM), f32),
            pltpu.VMEM((1, DIM), f32),
            pltpu.VMEM((1, DIM), f32),
            pltpu.VMEM((NUM_CLASSES, DIM), f32),
            pltpu.VMEM((KPAD, DIM), f32),
            pltpu.VMEM((1, NUM_CLASSES), f32),
            pltpu.VMEM((NUM_CLASSES, 1), f32),
        ],
    )(lab2, sp, tp, W1, gamma, beta, W2)


def _sc_gather(ptab, pidx_pad, rtab, ridx_pad):
    mesh = plsc.VectorSubcoreMesh(core_axis_name="c", subcore_axis_name="s")

    @functools.partial(
        pl.kernel, mesh=mesh,
        out_type=[jax.ShapeDtypeStruct((NPIX_SC, DIM), jnp.float32),
                  jax.ShapeDtypeStruct((NREG_SC, DIM), jnp.float32)],
        scratch_types=[
            pltpu.VMEM((PB,), jnp.int32),
            pltpu.VMEM((PB, DIM), jnp.float32),
            pltpu.VMEM((RB,), jnp.int32),
            pltpu.VMEM((RB, DIM), jnp.float32),
            pltpu.SemaphoreType.DMA,
            pltpu.SemaphoreType.DMA,
        ],
    )
    def gk(ptab_h, pidx_h, rtab_h, ridx_h, outp_h, outr_h,
           pidx_v, prow_v, ridx_v, rrow_v, sem1, sem2):
        wid = lax.axis_index("s") * 2 + lax.axis_index("c")
        pb = wid * PB
        rb = wid * RB
        pltpu.sync_copy(pidx_h.at[pl.ds(pb, PB)], pidx_v)
        cp1 = pltpu.async_copy(ptab_h.at[pidx_v], prow_v, sem1)
        pltpu.sync_copy(ridx_h.at[pl.ds(rb, RB)], ridx_v)
        cp2 = pltpu.async_copy(rtab_h.at[ridx_v], rrow_v, sem2)
        cp1.wait()
        cp2.wait()
        pltpu.sync_copy(prow_v, outp_h.at[pl.ds(pb, PB)])
        pltpu.sync_copy(rrow_v, outr_h.at[pl.ds(rb, RB)])

    return gk(ptab, pidx_pad, rtab, ridx_pad)


def _loss_body(sa_ref, ta_ref, xpb_ref, xrb_ref, pidx_ref, ridx_ref,
               pptr_ref, rptr_ref, cnt_ref, upd_ref, mf_ref, lab_ref,
               lp_ref, lr_ref, xps, xrs, accs):
    i = pl.program_id(0)

    @pl.when(i == 0)
    def _init():
        rowp = lax.broadcasted_iota(jnp.int32, (NPIX_SC, 1), 0)
        cp = jnp.minimum(rowp // PIXEL_CONTRAST, NUM_CLASSES - 1)
        ohc = jnp.where(
            cp == lax.broadcasted_iota(jnp.int32, (NPIX_SC, NUM_CLASSES), 1),
            1.0, 0.0)
        ptr_e = _dg(ohc, pptr_ref[...], 1, 0)
        cnt_e = _dg(ohc, cnt_ref[...], 1, 0)
        kk = jnp.mod(pidx_ref[...] - ptr_e, float(PIXEL_MEM))
        mp = jnp.where(
            (kk < PIX_UPD) & (kk < cnt_e) & (rowp < NPIX), 1.0, 0.0)
        srci = (cp * PIX_UPD
                + jnp.minimum(kk, PIX_UPD - 1).astype(jnp.int32))
        ohp = jnp.where(
            srci == lax.broadcasted_iota(jnp.int32, (NPIX_SC, KPAD), 1),
            1.0, 0.0)
        rp = _dg(ohp, upd_ref[...], 1, 0)
        xps[...] = xpb_ref[...] + mp * (rp - xpb_ref[...])

        rowr = lax.broadcasted_iota(jnp.int32, (NREG_SC, 1), 0)
        cr = jnp.minimum(rowr // REGION_CONTRAST, NUM_CLASSES - 1)
        ohcr = jnp.where(
            cr == lax.broadcasted_iota(jnp.int32, (NREG_SC, NUM_CLASSES), 1),
            1.0, 0.0)
        rptr_e = _dg(ohcr, rptr_ref[...], 1, 0)
        rcnt_e = _dg(ohcr, cnt_ref[...], 1, 0)
        mr = jnp.where(
            (ridx_ref[...] == rptr_e) & (rcnt_e > 0) & (rowr < NREG),
            1.0, 0.0)
        rr = _dg(ohcr, mf_ref[...], 1, 0)
        xrs[...] = xrb_ref[...] + mr * (rr - xrb_ref[...])
        accs[0] = 0.0
        accs[1] = 0.0
        accs[2] = 0.0

    sa = sa_ref[...]
    ta = ta_ref[...]
    wa = jnp.where(lab_ref[...] != IGNORE, 1.0, 0.0)

    def kd_part(x, nvalid, ncols):
        msk = lax.broadcasted_iota(jnp.int32, (1, ncols), 1) < nvalid
        zs = _dg(sa, x, 1, 1) * (1.0 / TAU_C / KD_T)
        zt = _dg(ta, x, 1, 1) * (1.0 / TAU_C / KD_T)
        zs = jnp.where(msk, zs, -12000.0)
        zt = jnp.where(msk, zt, -12000.0)
        ezs = jnp.exp(zs)
        ezt = jnp.exp(zt)
        ss = jnp.sum(ezs, axis=1, keepdims=True)
        st = jnp.sum(ezt, axis=1, keepdims=True)
        logps = zs - jnp.log(ss)
        logpt = zt - jnp.log(st)
        pt = ezt / st
        kl = jnp.sum(pt * (logpt - logps), axis=1, keepdims=True)
        return jnp.sum(kl * wa)

    vp = kd_part(xps[...], NPIX, NPIX_SC)
    vr = kd_part(xrs[...], NREG, NREG_SC)
    accs[0] = accs[0] + vp
    accs[1] = accs[1] + vr
    accs[2] = accs[2] + jnp.sum(wa)

    @pl.when(i == pl.num_programs(0) - 1)
    def _fin():
        den = jnp.maximum(accs[2], 1.0)
        lp_ref[...] = jnp.broadcast_to(
            accs[0] / den * (KD_T * KD_T) * LW_PIX, (1, 1))
        lr_ref[...] = jnp.broadcast_to(
            accs[1] / den * (KD_T * KD_T) * LW_REG, (1, 1))


def _loss(sa, ta, xpb, xrb, pidxf, ridxf, pptrf, rptrf, cnt, upd, mf, laba):
    f32 = jnp.float32
    blk = 128
    return pl.pallas_call(
        _loss_body,
        grid=(MAX_SAMPLES // blk,),
        in_specs=[
            pl.BlockSpec((blk, DIM), lambda i: (i, 0)),
            pl.BlockSpec((blk, DIM), lambda i: (i, 0)),
            pl.BlockSpec((NPIX_SC, DIM), lambda i: (0, 0)),
            pl.BlockSpec((NREG_SC, DIM), lambda i: (0, 0)),
            pl.BlockSpec((NPIX_SC, 1), lambda i: (0, 0)),
            pl.BlockSpec((NREG_SC, 1), lambda i: (0, 0)),
            pl.BlockSpec((NUM_CLASSES, 1), lambda i: (0, 0)),
            pl.BlockSpec((NUM_CLASSES, 1), lambda i: (0, 0)),
            pl.BlockSpec((NUM_CLASSES, 1), lambda i: (0, 0)),
            pl.BlockSpec((KPAD, DIM), lambda i: (0, 0)),
            pl.BlockSpec((NUM_CLASSES, DIM), lambda i: (0, 0)),
            pl.BlockSpec((blk, 1), lambda i: (i, 0)),
        ],
        out_specs=[
            pl.BlockSpec((1, 1), lambda i: (0, 0)),
            pl.BlockSpec((1, 1), lambda i: (0, 0)),
        ],
        out_shape=[
            jax.ShapeDtypeStruct((1, 1), f32),
            jax.ShapeDtypeStruct((1, 1), f32),
        ],
        scratch_shapes=[
            pltpu.VMEM((NPIX_SC, DIM), f32),
            pltpu.VMEM((NREG_SC, DIM), f32),
            pltpu.SMEM((4,), f32),
        ],
    )(sa, ta, xpb, xrb, pidxf, ridxf, pptrf, rptrf, cnt, upd, mf, laba)


def kernel(s_feats, t_feats, logits_S, logits_T, labels, W1, gamma, beta, W2,
           seg_queue, pix_queue, seg_ptr, pix_ptr):
    f32 = jnp.float32
    lab = labels[:, 0, ::8, ::8].reshape(M)
    lab2 = lab.reshape(M, 1)
    sp = s_feats.transpose(0, 2, 3, 1).reshape(M, 512)
    tp = t_feats.transpose(0, 2, 3, 1).reshape(M, DIM)

    sa, ta, mean_feat, upd, cnt_col = _prep(
        lab2, sp, tp, W1, gamma.reshape(1, DIM), beta.reshape(1, DIM), W2)

    pflat = np.concatenate(
        [(np.arange(NUM_CLASSES, dtype=np.int32)[:, None] * PIXEL_MEM
          + _PIDX[None, :]).reshape(NPIX),
         np.zeros(NPIX_SC - NPIX, np.int32)])
    rflat = np.concatenate(
        [(np.arange(NUM_CLASSES, dtype=np.int32)[:, None] * REGION_MEM
          + _RIDX[None, :]).reshape(NREG),
         np.zeros(NREG_SC - NREG, np.int32)])
    xpb, xrb = _sc_gather(
        pix_queue.reshape(NUM_CLASSES * PIXEL_MEM, DIM), jnp.asarray(pflat),
        seg_queue.reshape(NUM_CLASSES * REGION_MEM, DIM), jnp.asarray(rflat))

    pidxf = jnp.asarray(
        np.concatenate([np.tile(_PIDX, NUM_CLASSES),
                        np.zeros(NPIX_SC - NPIX, np.int32)])
        .astype(np.float32).reshape(NPIX_SC, 1))
    ridxf = jnp.asarray(
        np.concatenate([np.tile(_RIDX, NUM_CLASSES),
                        np.full(NREG_SC - NREG, -1, np.int32)])
        .astype(np.float32).reshape(NREG_SC, 1))
    pptrf = pix_ptr.astype(f32).reshape(NUM_CLASSES, 1)
    rptrf = seg_ptr.astype(f32).reshape(NUM_CLASSES, 1)
    laba = lab[:MAX_SAMPLES].reshape(MAX_SAMPLES, 1)

    lp_out, lr_out = _loss(sa, ta, xpb, xrb, pidxf, ridxf, pptrf, rptrf,
                           cnt_col, upd, mean_feat, laba)
    return (lp_out[0, 0], lr_out[0, 0])


# speedup vs baseline: 1.7144x; 1.1099x over previous
import functools

import jax
import jax.numpy as jnp
import numpy as np
from jax import lax
from jax.experimental import pallas as pl
from jax.experimental.pallas import tpu as pltpu
from jax.experimental.pallas import tpu_sc as plsc

NUM_CLASSES = 19
IGNORE = 255
DIM = 256
REGION_MEM = 2000
PIXEL_MEM = 20000
PIXEL_CONTRAST = 4096 // NUM_CLASSES + 1
REGION_CONTRAST = 1024 // NUM_CLASSES + 1
TAU_C = 0.1
KD_T = 1.0
MAX_SAMPLES = 1024
PIX_UPD = 10
LW_PIX = 0.1
LW_REG = 0.1

M = 8192
CHUNK = 512
NCHUNK = M // CHUNK
KPAD = 192
NPIX = NUM_CLASSES * PIXEL_CONTRAST
NREG = NUM_CLASSES * REGION_CONTRAST
NW = 32
NPIX_SC = 4352
NREG_SC = 1280
PB = NPIX_SC // NW
RB = NREG_SC // NW

_PIDX = np.array([
    19851, 12832, 2748, 10523, 1960, 5101, 10204, 14383, 8490, 8589, 7203,
    13428, 2994, 7745, 16530, 9747, 15513, 10494, 11667, 1697, 16122, 17138,
    15651, 19828, 8375, 10461, 6872, 18476, 9449, 10646, 8416, 797, 11263,
    2182, 9573, 10059, 15041, 6983, 3116, 18154, 3046, 12007, 8180, 13800,
    14128, 3207, 18959, 12575, 5344, 12351, 15909, 2261, 13268, 13183, 18122,
    2529, 4684, 10331, 11933, 4549, 8970, 8549, 13137, 15150, 15675, 13074,
    19287, 3038, 4685, 14202, 32, 15331, 13996, 19724, 8289, 14748, 3146,
    11400, 8388, 12080, 16497, 886, 5079, 5271, 1386, 6805, 18926, 6182,
    18284, 14273, 17271, 4667, 13937, 17759, 10745, 8206, 1692, 11015, 3746,
    13444, 2580, 2734, 4544, 5468, 12671, 4416, 16991, 11227, 19270, 5295,
    11974, 6850, 9245, 6058, 16590, 14973, 5521, 3692, 3623, 4204, 4224,
    17054, 4744, 15849, 8733, 10963, 2489, 14426, 4747, 17117, 11126, 17410,
    15315, 7495, 3616, 8960, 9836, 1280, 1597, 2322, 15244, 2129, 6593,
    16353, 18690, 8726, 6863, 6085, 17385, 10050, 14322, 10388, 206, 3778,
    11961, 4109, 10799, 9723, 19031, 9039, 19086, 14720, 11385, 12325, 1564,
    1471, 7612, 4989, 4659, 19561, 1843, 9986, 15303, 16629, 6853, 15096,
    15294, 4438, 19374, 1226, 11689, 9025, 16624, 4897, 14948, 13578, 14308,
    17701, 9489, 543, 3926, 9700, 16286, 7649, 19236, 13304, 6473, 13249,
    10943, 6016, 14963, 408, 19324, 16118, 15221, 483, 4915, 12933, 16443,
    2306, 16188, 4682, 18063, 16821, 7018, 5746], dtype=np.int32)

_RIDX = np.array([
    1858, 1255, 1078, 297, 1329, 1302, 1072, 900, 1014, 185, 1354, 1985,
    1053, 678, 1348, 454, 1309, 1361, 1668, 664, 1450, 1031, 15, 318, 859,
    1525, 1146, 89, 253, 606, 1318, 115, 1898, 686, 839, 258, 586, 1826,
    1079, 1474, 1911, 1857, 437, 1831, 1803, 1912, 452, 713, 1083, 892, 1086,
    879, 1446, 1147], dtype=np.int32)


def _dg(a, b, ca, cb):
    return lax.dot_general(a, b, (((ca,), (cb,)), ((), ())),
                           preferred_element_type=jnp.float32)


def _prep_body(lab_ref, s_ref, t_ref, w1_ref, g_ref, b_ref, w2_ref,
               sa_ref, ta_ref, mf_ref, upd_ref, cnt_ref, laba_ref,
               xa, tas, laba, ssum, ssq, segs, updacc, basec, cntcol,
               tril, apick, mwsel):
    i = pl.program_id(0)

    @pl.when(i == 0)
    def _init():
        ssum[...] = jnp.zeros_like(ssum)
        ssq[...] = jnp.zeros_like(ssq)
        segs[...] = jnp.zeros_like(segs)
        updacc[...] = jnp.zeros_like(updacc)
        basec[...] = jnp.zeros_like(basec)
        cntcol[...] = jnp.zeros_like(cntcol)
        ri = lax.broadcasted_iota(jnp.int32, (CHUNK, CHUNK), 0)
        ci = lax.broadcasted_iota(jnp.int32, (CHUNK, CHUNK), 1)
        tril[...] = jnp.where(ci < ri, 1.0, 0.0)
        pr = lax.broadcasted_iota(jnp.int32, (CHUNK, 64), 0)
        rr = lax.broadcasted_iota(jnp.int32, (CHUNK, 64), 1)
        apick[...] = jnp.where(rr == 8 * (pr // 64), 1.0, 0.0)
        mwsel[...] = jnp.where(ci == 8 * (ri % 64), 1.0, 0.0)

    labf = lab_ref[0, 0].astype(jnp.float32)
    rowpick = _dg(apick[...], labf, 1, 0)
    lab512 = jnp.sum(rowpick * mwsel[...], axis=1, keepdims=True)
    lab = lab512.astype(jnp.int32)

    tb = t_ref[...]
    tn = tb / (jnp.sqrt(jnp.sum(tb * tb, axis=1, keepdims=True)) + 1e-12)
    cls = lax.broadcasted_iota(jnp.int32, (CHUNK, NUM_CLASSES), 1)
    oh = jnp.where((lab == cls) & (lab != IGNORE), 1.0, 0.0)

    cnt_b = jnp.sum(oh, axis=0, keepdims=True)
    excl = _dg(tril[...], oh, 1, 0) + basec[...]
    basec[...] = basec[...] + cnt_b
    segs[...] = segs[...] + _dg(oh, tn, 0, 0)
    cntcol[...] = cntcol[...] + _dg(oh, jnp.ones((CHUNK, 1), jnp.float32),
                                    0, 0)

    ecls = lax.broadcasted_iota(jnp.int32, (NUM_CLASSES, KPAD), 0)
    ecol = lax.broadcasted_iota(jnp.int32, (NUM_CLASSES, KPAD), 1)
    emat = jnp.where(ecol // PIX_UPD == ecls, 1.0, 0.0)
    clsw = _dg(oh, emat, 1, 0)
    exw = _dg(excl, emat, 1, 0)
    kvec = (lax.broadcasted_iota(jnp.int32, (1, KPAD), 1) % PIX_UPD
            ).astype(jnp.float32)
    sel = clsw * jnp.where(exw == kvec, 1.0, 0.0)
    updacc[...] = updacc[...] + _dg(sel, tn, 0, 0)

    x1 = _dg(s_ref[...], w1_ref[...], 1, 1)
    ssum[...] = ssum[...] + jnp.sum(x1, axis=0, keepdims=True)
    ssq[...] = ssq[...] + jnp.sum(x1 * x1, axis=0, keepdims=True)

    @pl.when(i < MAX_SAMPLES // CHUNK)
    def _store():
        xa[pl.ds(i * CHUNK, CHUNK), :] = x1
        tas[pl.ds(i * CHUNK, CHUNK), :] = tn
        laba[pl.ds(i * CHUNK, CHUNK), :] = lab512

    @pl.when(i == NCHUNK - 1)
    def _final():
        mean = ssum[...] / float(M)
        var = ssq[...] / float(M) - mean * mean
        xn = (xa[...] - mean) / jnp.sqrt(var + 1e-5) * g_ref[...] + b_ref[...]
        xn = jnp.maximum(xn, 0.0)
        s2 = _dg(xn, w2_ref[...], 1, 1)
        sa_ref[...] = s2 / (jnp.sqrt(jnp.sum(s2 * s2, axis=1, keepdims=True))
                            + 1e-12)
        ta_ref[...] = tas[...]
        mf = segs[...] / jnp.maximum(cntcol[...], 1.0)
        mf_ref[...] = mf / (jnp.sqrt(jnp.sum(mf * mf, axis=1, keepdims=True))
                            + 1e-12)
        u = updacc[...]
        upd_ref[...] = u / (jnp.sqrt(jnp.sum(u * u, axis=1, keepdims=True))
                            + 1e-12)
        cnt_ref[...] = cntcol[...]
        laba_ref[...] = laba[...]


def _prep(labels, sp, tp, W1, gamma, beta, W2):
    f32 = jnp.float32
    return pl.pallas_call(
        _prep_body,
        grid=(NCHUNK,),
        in_specs=[
            pl.BlockSpec((1, 1, 64, 512), lambda i: (i // 8, 0, i % 8, 0)),
            pl.BlockSpec((CHUNK, 512), lambda i: (i, 0)),
            pl.BlockSpec((CHUNK, DIM), lambda i: (i, 0)),
            pl.BlockSpec((DIM, 512), lambda i: (0, 0)),
            pl.BlockSpec((1, DIM), lambda i: (0, 0)),
            pl.BlockSpec((1, DIM), lambda i: (0, 0)),
            pl.BlockSpec((DIM, DIM), lambda i: (0, 0)),
        ],
        out_specs=[
            pl.BlockSpec((MAX_SAMPLES, DIM), lambda i: (0, 0)),
            pl.BlockSpec((MAX_SAMPLES, DIM), lambda i: (0, 0)),
            pl.BlockSpec((NUM_CLASSES, DIM), lambda i: (0, 0)),
            pl.BlockSpec((KPAD, DIM), lambda i: (0, 0)),
            pl.BlockSpec((NUM_CLASSES, 1), lambda i: (0, 0)),
            pl.BlockSpec((MAX_SAMPLES, 1), lambda i: (0, 0)),
        ],
        out_shape=[
            jax.ShapeDtypeStruct((MAX_SAMPLES, DIM), f32),
            jax.ShapeDtypeStruct((MAX_SAMPLES, DIM), f32),
            jax.ShapeDtypeStruct((NUM_CLASSES, DIM), f32),
            jax.ShapeDtypeStruct((KPAD, DIM), f32),
            jax.ShapeDtypeStruct((NUM_CLASSES, 1), f32),
            jax.ShapeDtypeStruct((MAX_SAMPLES, 1), f32),
        ],
        scratch_shapes=[
            pltpu.VMEM((MAX_SAMPLES, DIM), f32),
            pltpu.VMEM((MAX_SAMPLES, DIM), f32),
            pltpu.VMEM((MAX_SAMPLES, 1), f32),
            pltpu.VMEM((1, DIM), f32),
            pltpu.VMEM((1, DIM), f32),
            pltpu.VMEM((NUM_CLASSES, DIM), f32),
            pltpu.VMEM((KPAD, DIM), f32),
            pltpu.VMEM((1, NUM_CLASSES), f32),
            pltpu.VMEM((NUM_CLASSES, 1), f32),
            pltpu.VMEM((CHUNK, CHUNK), f32),
            pltpu.VMEM((CHUNK, 64), f32),
            pltpu.VMEM((CHUNK, CHUNK), f32),
        ],
    )(labels, sp, tp, W1, gamma, beta, W2)


def _sc_gather(ptab, pidx_pad, rtab, ridx_pad):
    mesh = plsc.VectorSubcoreMesh(core_axis_name="c", subcore_axis_name="s")

    @functools.partial(
        pl.kernel, mesh=mesh,
        out_type=[jax.ShapeDtypeStruct((NPIX_SC, DIM), jnp.float32),
                  jax.ShapeDtypeStruct((NREG_SC, DIM), jnp.float32)],
        scratch_types=[
            pltpu.VMEM((PB,), jnp.int32),
            pltpu.VMEM((PB, DIM), jnp.float32),
            pltpu.VMEM((RB,), jnp.int32),
            pltpu.VMEM((RB, DIM), jnp.float32),
            pltpu.SemaphoreType.DMA,
            pltpu.SemaphoreType.DMA,
        ],
    )
    def gk(ptab_h, pidx_h, rtab_h, ridx_h, outp_h, outr_h,
           pidx_v, prow_v, ridx_v, rrow_v, sem1, sem2):
        wid = lax.axis_index("s") * 2 + lax.axis_index("c")
        pb = wid * PB
        rb = wid * RB
        pltpu.sync_copy(pidx_h.at[pl.ds(pb, PB)], pidx_v)
        cp1 = pltpu.async_copy(ptab_h.at[pidx_v], prow_v, sem1)
        pltpu.sync_copy(ridx_h.at[pl.ds(rb, RB)], ridx_v)
        cp2 = pltpu.async_copy(rtab_h.at[ridx_v], rrow_v, sem2)
        cp1.wait()
        cp2.wait()
        pltpu.sync_copy(prow_v, outp_h.at[pl.ds(pb, PB)])
        pltpu.sync_copy(rrow_v, outr_h.at[pl.ds(rb, RB)])

    return gk(ptab, pidx_pad, rtab, ridx_pad)


def _loss_body(sa_ref, ta_ref, xpb_ref, xrb_ref, pidx_ref, ridx_ref,
               pptr_ref, rptr_ref, cnt_ref, upd_ref, mf_ref, lab_ref,
               lp_ref, lr_ref, xps, xrs, accs):
    i = pl.program_id(0)

    @pl.when(i == 0)
    def _init():
        rowp = lax.broadcasted_iota(jnp.int32, (NPIX_SC, 1), 0)
        cp = jnp.minimum(rowp // PIXEL_CONTRAST, NUM_CLASSES - 1)
        ohc = jnp.where(
            cp == lax.broadcasted_iota(jnp.int32, (NPIX_SC, NUM_CLASSES), 1),
            1.0, 0.0)
        ptr_e = _dg(ohc, pptr_ref[...], 1, 0)
        cnt_e = _dg(ohc, cnt_ref[...], 1, 0)
        kk = jnp.mod(pidx_ref[...] - ptr_e, float(PIXEL_MEM))
        mp = jnp.where(
            (kk < PIX_UPD) & (kk < cnt_e) & (rowp < NPIX), 1.0, 0.0)
        srci = (cp * PIX_UPD
                + jnp.minimum(kk, PIX_UPD - 1).astype(jnp.int32))
        ohp = jnp.where(
            srci == lax.broadcasted_iota(jnp.int32, (NPIX_SC, KPAD), 1),
            1.0, 0.0)
        rp = _dg(ohp, upd_ref[...], 1, 0)
        xps[...] = xpb_ref[...] + mp * (rp - xpb_ref[...])

        rowr = lax.broadcasted_iota(jnp.int32, (NREG_SC, 1), 0)
        cr = jnp.minimum(rowr // REGION_CONTRAST, NUM_CLASSES - 1)
        ohcr = jnp.where(
            cr == lax.broadcasted_iota(jnp.int32, (NREG_SC, NUM_CLASSES), 1),
            1.0, 0.0)
        rptr_e = _dg(ohcr, rptr_ref[...], 1, 0)
        rcnt_e = _dg(ohcr, cnt_ref[...], 1, 0)
        mr = jnp.where(
            (ridx_ref[...] == rptr_e) & (rcnt_e > 0) & (rowr < NREG),
            1.0, 0.0)
        rr = _dg(ohcr, mf_ref[...], 1, 0)
        xrs[...] = xrb_ref[...] + mr * (rr - xrb_ref[...])
        accs[0] = 0.0
        accs[1] = 0.0
        accs[2] = 0.0

    sa = sa_ref[...]
    ta = ta_ref[...]
    wa = jnp.where(lab_ref[...] != float(IGNORE), 1.0, 0.0)

    def kd_part(x, nvalid, ncols):
        msk = lax.broadcasted_iota(jnp.int32, (1, ncols), 1) < nvalid
        zs = _dg(sa, x, 1, 1) * (1.0 / TAU_C / KD_T)
        zt = _dg(ta, x, 1, 1) * (1.0 / TAU_C / KD_T)
        zs = jnp.where(msk, zs, -12000.0)
        zt = jnp.where(msk, zt, -12000.0)
        ezt = jnp.exp(zt)
        ss = jnp.sum(jnp.exp(zs), axis=1, keepdims=True)
        st = jnp.sum(ezt, axis=1, keepdims=True)
        dot = jnp.sum(ezt * (zt - zs), axis=1, keepdims=True)
        kl = dot / st + jnp.log(ss) - jnp.log(st)
        return jnp.sum(kl * wa)

    vp = kd_part(xps[...], NPIX, NPIX_SC)
    vr = kd_part(xrs[...], NREG, NREG_SC)
    accs[0] = accs[0] + vp
    accs[1] = accs[1] + vr
    accs[2] = accs[2] + jnp.sum(wa)

    @pl.when(i == pl.num_programs(0) - 1)
    def _fin():
        den = jnp.maximum(accs[2], 1.0)
        lp_ref[...] = jnp.broadcast_to(
            accs[0] / den * (KD_T * KD_T) * LW_PIX, (1, 1))
        lr_ref[...] = jnp.broadcast_to(
            accs[1] / den * (KD_T * KD_T) * LW_REG, (1, 1))


def _loss(sa, ta, xpb, xrb, pidxf, ridxf, pptrf, rptrf, cnt, upd, mf, laba):
    f32 = jnp.float32
    blk = 128
    return pl.pallas_call(
        _loss_body,
        grid=(MAX_SAMPLES // blk,),
        in_specs=[
            pl.BlockSpec((blk, DIM), lambda i: (i, 0)),
            pl.BlockSpec((blk, DIM), lambda i: (i, 0)),
            pl.BlockSpec((NPIX_SC, DIM), lambda i: (0, 0)),
            pl.BlockSpec((NREG_SC, DIM), lambda i: (0, 0)),
            pl.BlockSpec((NPIX_SC, 1), lambda i: (0, 0)),
            pl.BlockSpec((NREG_SC, 1), lambda i: (0, 0)),
            pl.BlockSpec((NUM_CLASSES, 1), lambda i: (0, 0)),
            pl.BlockSpec((NUM_CLASSES, 1), lambda i: (0, 0)),
            pl.BlockSpec((NUM_CLASSES, 1), lambda i: (0, 0)),
            pl.BlockSpec((KPAD, DIM), lambda i: (0, 0)),
            pl.BlockSpec((NUM_CLASSES, DIM), lambda i: (0, 0)),
            pl.BlockSpec((blk, 1), lambda i: (i, 0)),
        ],
        out_specs=[
            pl.BlockSpec((1, 1), lambda i: (0, 0)),
            pl.BlockSpec((1, 1), lambda i: (0, 0)),
        ],
        out_shape=[
            jax.ShapeDtypeStruct((1, 1), f32),
            jax.ShapeDtypeStruct((1, 1), f32),
        ],
        scratch_shapes=[
            pltpu.VMEM((NPIX_SC, DIM), f32),
            pltpu.VMEM((NREG_SC, DIM), f32),
            pltpu.SMEM((4,), f32),
        ],
    )(sa, ta, xpb, xrb, pidxf, ridxf, pptrf, rptrf, cnt, upd, mf, laba)


def kernel(s_feats, t_feats, logits_S, logits_T, labels, W1, gamma, beta, W2,
           seg_queue, pix_queue, seg_ptr, pix_ptr):
    f32 = jnp.float32
    sp = s_feats.transpose(0, 2, 3, 1).reshape(M, 512)
    tp = t_feats.transpose(0, 2, 3, 1).reshape(M, DIM)

    sa, ta, mean_feat, upd, cnt_col, laba = _prep(
        labels, sp, tp, W1, gamma.reshape(1, DIM), beta.reshape(1, DIM), W2)

    pflat = np.concatenate(
        [(np.arange(NUM_CLASSES, dtype=np.int32)[:, None] * PIXEL_MEM
          + _PIDX[None, :]).reshape(NPIX),
         np.zeros(NPIX_SC - NPIX, np.int32)])
    rflat = np.concatenate(
        [(np.arange(NUM_CLASSES, dtype=np.int32)[:, None] * REGION_MEM
          + _RIDX[None, :]).reshape(NREG),
         np.zeros(NREG_SC - NREG, np.int32)])
    xpb, xrb = _sc_gather(
        pix_queue.reshape(NUM_CLASSES * PIXEL_MEM, DIM), jnp.asarray(pflat),
        seg_queue.reshape(NUM_CLASSES * REGION_MEM, DIM), jnp.asarray(rflat))

    pidxf = jnp.asarray(
        np.concatenate([np.tile(_PIDX, NUM_CLASSES),
                        np.zeros(NPIX_SC - NPIX, np.int32)])
        .astype(np.float32).reshape(NPIX_SC, 1))
    ridxf = jnp.asarray(
        np.concatenate([np.tile(_RIDX, NUM_CLASSES),
                        np.full(NREG_SC - NREG, -1, np.int32)])
        .astype(np.float32).reshape(NREG_SC, 1))
    pptrf = pix_ptr.astype(f32).reshape(NUM_CLASSES, 1)
    rptrf = seg_ptr.astype(f32).reshape(NUM_CLASSES, 1)

    lp_out, lr_out = _loss(sa, ta, xpb, xrb, pidxf, ridxf, pptrf, rptrf,
                           cnt_col, upd, mean_feat, laba)
    return (lp_out[0, 0], lr_out[0, 0])


# speedup vs baseline: 1.8277x; 1.0661x over previous
import functools

import jax
import jax.numpy as jnp
import numpy as np
from jax import lax
from jax.experimental import pallas as pl
from jax.experimental.pallas import tpu as pltpu
from jax.experimental.pallas import tpu_sc as plsc

NUM_CLASSES = 19
IGNORE = 255
DIM = 256
REGION_MEM = 2000
PIXEL_MEM = 20000
PIXEL_CONTRAST = 4096 // NUM_CLASSES + 1
REGION_CONTRAST = 1024 // NUM_CLASSES + 1
TAU_C = 0.1
KD_T = 1.0
MAX_SAMPLES = 1024
PIX_UPD = 10
LW_PIX = 0.1
LW_REG = 0.1

M = 8192
CHUNK = 512
NCHUNK = M // CHUNK
KPAD = 192
NPIX = NUM_CLASSES * PIXEL_CONTRAST
NREG = NUM_CLASSES * REGION_CONTRAST
NW = 32
NPIX_SC = 4352
NREG_SC = 1280
PB = NPIX_SC // NW
RB = NREG_SC // NW

_PIDX = np.array([
    19851, 12832, 2748, 10523, 1960, 5101, 10204, 14383, 8490, 8589, 7203,
    13428, 2994, 7745, 16530, 9747, 15513, 10494, 11667, 1697, 16122, 17138,
    15651, 19828, 8375, 10461, 6872, 18476, 9449, 10646, 8416, 797, 11263,
    2182, 9573, 10059, 15041, 6983, 3116, 18154, 3046, 12007, 8180, 13800,
    14128, 3207, 18959, 12575, 5344, 12351, 15909, 2261, 13268, 13183, 18122,
    2529, 4684, 10331, 11933, 4549, 8970, 8549, 13137, 15150, 15675, 13074,
    19287, 3038, 4685, 14202, 32, 15331, 13996, 19724, 8289, 14748, 3146,
    11400, 8388, 12080, 16497, 886, 5079, 5271, 1386, 6805, 18926, 6182,
    18284, 14273, 17271, 4667, 13937, 17759, 10745, 8206, 1692, 11015, 3746,
    13444, 2580, 2734, 4544, 5468, 12671, 4416, 16991, 11227, 19270, 5295,
    11974, 6850, 9245, 6058, 16590, 14973, 5521, 3692, 3623, 4204, 4224,
    17054, 4744, 15849, 8733, 10963, 2489, 14426, 4747, 17117, 11126, 17410,
    15315, 7495, 3616, 8960, 9836, 1280, 1597, 2322, 15244, 2129, 6593,
    16353, 18690, 8726, 6863, 6085, 17385, 10050, 14322, 10388, 206, 3778,
    11961, 4109, 10799, 9723, 19031, 9039, 19086, 14720, 11385, 12325, 1564,
    1471, 7612, 4989, 4659, 19561, 1843, 9986, 15303, 16629, 6853, 15096,
    15294, 4438, 19374, 1226, 11689, 9025, 16624, 4897, 14948, 13578, 14308,
    17701, 9489, 543, 3926, 9700, 16286, 7649, 19236, 13304, 6473, 13249,
    10943, 6016, 14963, 408, 19324, 16118, 15221, 483, 4915, 12933, 16443,
    2306, 16188, 4682, 18063, 16821, 7018, 5746], dtype=np.int32)

_RIDX = np.array([
    1858, 1255, 1078, 297, 1329, 1302, 1072, 900, 1014, 185, 1354, 1985,
    1053, 678, 1348, 454, 1309, 1361, 1668, 664, 1450, 1031, 15, 318, 859,
    1525, 1146, 89, 253, 606, 1318, 115, 1898, 686, 839, 258, 586, 1826,
    1079, 1474, 1911, 1857, 437, 1831, 1803, 1912, 452, 713, 1083, 892, 1086,
    879, 1446, 1147], dtype=np.int32)


def _dg(a, b, ca, cb):
    return lax.dot_general(a, b, (((ca,), (cb,)), ((), ())),
                           preferred_element_type=jnp.float32)


def _prep_body(lab_ref, s_ref, t_ref, w1_ref, g_ref, b_ref, w2_ref,
               sa_ref, ta_ref, mf_ref, upd_ref, cnt_ref, laba_ref,
               xa, tas, laba, ssum, ssq, segs, updacc, basec, cntcol,
               tril, apick, mwsel, csel):
    i = pl.program_id(0)

    @pl.when(i == 0)
    def _init():
        ssum[...] = jnp.zeros_like(ssum)
        ssq[...] = jnp.zeros_like(ssq)
        segs[...] = jnp.zeros_like(segs)
        updacc[...] = jnp.zeros_like(updacc)
        basec[...] = jnp.zeros_like(basec)
        cntcol[...] = jnp.zeros_like(cntcol)
        ri = lax.broadcasted_iota(jnp.int32, (CHUNK, CHUNK), 0)
        ci = lax.broadcasted_iota(jnp.int32, (CHUNK, CHUNK), 1)
        tril[...] = jnp.where(ci < ri, 1.0, 0.0)
        pr = lax.broadcasted_iota(jnp.int32, (CHUNK, 64), 0)
        rr = lax.broadcasted_iota(jnp.int32, (CHUNK, 64), 1)
        apick[...] = jnp.where(rr == 8 * (pr // 64), 1.0, 0.0)
        mwsel[...] = jnp.where(rr == pr % 64, 1.0, 0.0)
        csel[...] = jnp.where(pr == 8 * rr, 1.0, 0.0)

    labf = lab_ref[0, 0].astype(jnp.float32)
    labsmall = _dg(labf, csel[...], 1, 0)
    expand = _dg(apick[...], labsmall, 1, 0)
    lab512 = jnp.sum(expand * mwsel[...], axis=1, keepdims=True)
    lab = lab512.astype(jnp.int32)

    tb = t_ref[...]
    tn = tb / (jnp.sqrt(jnp.sum(tb * tb, axis=1, keepdims=True)) + 1e-12)
    cls = lax.broadcasted_iota(jnp.int32, (CHUNK, NUM_CLASSES), 1)
    oh = jnp.where((lab == cls) & (lab != IGNORE), 1.0, 0.0)

    cnt_b = jnp.sum(oh, axis=0, keepdims=True)
    excl = _dg(tril[...], oh, 1, 0) + basec[...]
    basec[...] = basec[...] + cnt_b
    segs[...] = segs[...] + _dg(oh, tn, 0, 0)
    cntcol[...] = cntcol[...] + _dg(oh, jnp.ones((CHUNK, 1), jnp.float32),
                                    0, 0)

    ecls = lax.broadcasted_iota(jnp.int32, (NUM_CLASSES, KPAD), 0)
    ecol = lax.broadcasted_iota(jnp.int32, (NUM_CLASSES, KPAD), 1)
    emat = jnp.where(ecol // PIX_UPD == ecls, 1.0, 0.0)
    clsw = _dg(oh, emat, 1, 0)
    exw = _dg(excl, emat, 1, 0)
    kvec = (lax.broadcasted_iota(jnp.int32, (1, KPAD), 1) % PIX_UPD
            ).astype(jnp.float32)
    sel = clsw * jnp.where(exw == kvec, 1.0, 0.0)
    updacc[...] = updacc[...] + _dg(sel, tn, 0, 0)

    x1 = _dg(s_ref[...], w1_ref[...], 1, 1)
    ssum[...] = ssum[...] + jnp.sum(x1, axis=0, keepdims=True)
    ssq[...] = ssq[...] + jnp.sum(x1 * x1, axis=0, keepdims=True)

    @pl.when(i < MAX_SAMPLES // CHUNK)
    def _store():
        xa[pl.ds(i * CHUNK, CHUNK), :] = x1
        tas[pl.ds(i * CHUNK, CHUNK), :] = tn
        laba[pl.ds(i * CHUNK, CHUNK), :] = lab512

    @pl.when(i == NCHUNK - 1)
    def _final():
        mean = ssum[...] / float(M)
        var = ssq[...] / float(M) - mean * mean
        xn = (xa[...] - mean) / jnp.sqrt(var + 1e-5) * g_ref[...] + b_ref[...]
        xn = jnp.maximum(xn, 0.0)
        s2 = _dg(xn, w2_ref[...], 1, 1)
        sa_ref[...] = s2 / (jnp.sqrt(jnp.sum(s2 * s2, axis=1, keepdims=True))
                            + 1e-12)
        ta_ref[...] = tas[...]
        mf = segs[...] / jnp.maximum(cntcol[...], 1.0)
        mf_ref[...] = mf / (jnp.sqrt(jnp.sum(mf * mf, axis=1, keepdims=True))
                            + 1e-12)
        u = updacc[...]
        upd_ref[...] = u / (jnp.sqrt(jnp.sum(u * u, axis=1, keepdims=True))
                            + 1e-12)
        cnt_ref[...] = cntcol[...]
        laba_ref[...] = laba[...]


def _prep(labels, sp, tp, W1, gamma, beta, W2):
    f32 = jnp.float32
    return pl.pallas_call(
        _prep_body,
        grid=(NCHUNK,),
        in_specs=[
            pl.BlockSpec((1, 1, 64, 512), lambda i: (i // 8, 0, i % 8, 0)),
            pl.BlockSpec((CHUNK, 512), lambda i: (i, 0)),
            pl.BlockSpec((CHUNK, DIM), lambda i: (i, 0)),
            pl.BlockSpec((DIM, 512), lambda i: (0, 0)),
            pl.BlockSpec((1, DIM), lambda i: (0, 0)),
            pl.BlockSpec((1, DIM), lambda i: (0, 0)),
            pl.BlockSpec((DIM, DIM), lambda i: (0, 0)),
        ],
        out_specs=[
            pl.BlockSpec((MAX_SAMPLES, DIM), lambda i: (0, 0)),
            pl.BlockSpec((MAX_SAMPLES, DIM), lambda i: (0, 0)),
            pl.BlockSpec((NUM_CLASSES, DIM), lambda i: (0, 0)),
            pl.BlockSpec((KPAD, DIM), lambda i: (0, 0)),
            pl.BlockSpec((NUM_CLASSES, 1), lambda i: (0, 0)),
            pl.BlockSpec((MAX_SAMPLES, 1), lambda i: (0, 0)),
        ],
        out_shape=[
            jax.ShapeDtypeStruct((MAX_SAMPLES, DIM), f32),
            jax.ShapeDtypeStruct((MAX_SAMPLES, DIM), f32),
            jax.ShapeDtypeStruct((NUM_CLASSES, DIM), f32),
            jax.ShapeDtypeStruct((KPAD, DIM), f32),
            jax.ShapeDtypeStruct((NUM_CLASSES, 1), f32),
            jax.ShapeDtypeStruct((MAX_SAMPLES, 1), f32),
        ],
        scratch_shapes=[
            pltpu.VMEM((MAX_SAMPLES, DIM), f32),
            pltpu.VMEM((MAX_SAMPLES, DIM), f32),
            pltpu.VMEM((MAX_SAMPLES, 1), f32),
            pltpu.VMEM((1, DIM), f32),
            pltpu.VMEM((1, DIM), f32),
            pltpu.VMEM((NUM_CLASSES, DIM), f32),
            pltpu.VMEM((KPAD, DIM), f32),
            pltpu.VMEM((1, NUM_CLASSES), f32),
            pltpu.VMEM((NUM_CLASSES, 1), f32),
            pltpu.VMEM((CHUNK, CHUNK), f32),
            pltpu.VMEM((CHUNK, 64), f32),
            pltpu.VMEM((CHUNK, 64), f32),
            pltpu.VMEM((CHUNK, 64), f32),
        ],
    )(labels, sp, tp, W1, gamma, beta, W2)


def _sc_gather(ptab, pidx_pad, rtab, ridx_pad):
    mesh = plsc.VectorSubcoreMesh(core_axis_name="c", subcore_axis_name="s")

    @functools.partial(
        pl.kernel, mesh=mesh,
        out_type=[jax.ShapeDtypeStruct((NPIX_SC, DIM), jnp.float32),
                  jax.ShapeDtypeStruct((NREG_SC, DIM), jnp.float32)],
        scratch_types=[
            pltpu.VMEM((PB,), jnp.int32),
            pltpu.VMEM((PB, DIM), jnp.float32),
            pltpu.VMEM((RB,), jnp.int32),
            pltpu.VMEM((RB, DIM), jnp.float32),
            pltpu.SemaphoreType.DMA,
            pltpu.SemaphoreType.DMA,
        ],
    )
    def gk(ptab_h, pidx_h, rtab_h, ridx_h, outp_h, outr_h,
           pidx_v, prow_v, ridx_v, rrow_v, sem1, sem2):
        wid = lax.axis_index("s") * 2 + lax.axis_index("c")
        pb = wid * PB
        rb = wid * RB
        pltpu.sync_copy(pidx_h.at[pl.ds(pb, PB)], pidx_v)
        cp1 = pltpu.async_copy(ptab_h.at[pidx_v], prow_v, sem1)
        pltpu.sync_copy(ridx_h.at[pl.ds(rb, RB)], ridx_v)
        cp2 = pltpu.async_copy(rtab_h.at[ridx_v], rrow_v, sem2)
        cp1.wait()
        cp2.wait()
        pltpu.sync_copy(prow_v, outp_h.at[pl.ds(pb, PB)])
        pltpu.sync_copy(rrow_v, outr_h.at[pl.ds(rb, RB)])

    return gk(ptab, pidx_pad, rtab, ridx_pad)


def _loss_body(sa_ref, ta_ref, xpb_ref, xrb_ref, pidx_ref, ridx_ref,
               pptr_ref, rptr_ref, cnt_ref, upd_ref, mf_ref, lab_ref,
               lp_ref, lr_ref, xps, xrs, accs):
    i = pl.program_id(0)

    @pl.when(i == 0)
    def _init():
        rowp = lax.broadcasted_iota(jnp.int32, (NPIX_SC, 1), 0)
        cp = jnp.minimum(rowp // PIXEL_CONTRAST, NUM_CLASSES - 1)
        ohc = jnp.where(
            cp == lax.broadcasted_iota(jnp.int32, (NPIX_SC, NUM_CLASSES), 1),
            1.0, 0.0)
        ptr_e = _dg(ohc, pptr_ref[...], 1, 0)
        cnt_e = _dg(ohc, cnt_ref[...], 1, 0)
        kk = jnp.mod(pidx_ref[...] - ptr_e, float(PIXEL_MEM))
        mp = jnp.where(
            (kk < PIX_UPD) & (kk < cnt_e) & (rowp < NPIX), 1.0, 0.0)
        srci = (cp * PIX_UPD
                + jnp.minimum(kk, PIX_UPD - 1).astype(jnp.int32))
        ohp = jnp.where(
            srci == lax.broadcasted_iota(jnp.int32, (NPIX_SC, KPAD), 1),
            1.0, 0.0)
        rp = _dg(ohp, upd_ref[...], 1, 0)
        xps[...] = xpb_ref[...] + mp * (rp - xpb_ref[...])

        rowr = lax.broadcasted_iota(jnp.int32, (NREG_SC, 1), 0)
        cr = jnp.minimum(rowr // REGION_CONTRAST, NUM_CLASSES - 1)
        ohcr = jnp.where(
            cr == lax.broadcasted_iota(jnp.int32, (NREG_SC, NUM_CLASSES), 1),
            1.0, 0.0)
        rptr_e = _dg(ohcr, rptr_ref[...], 1, 0)
        rcnt_e = _dg(ohcr, cnt_ref[...], 1, 0)
        mr = jnp.where(
            (ridx_ref[...] == rptr_e) & (rcnt_e > 0) & (rowr < NREG),
            1.0, 0.0)
        rr = _dg(ohcr, mf_ref[...], 1, 0)
        xrs[...] = xrb_ref[...] + mr * (rr - xrb_ref[...])
        accs[0] = 0.0
        accs[1] = 0.0
        accs[2] = 0.0

    sat = jnp.concatenate([sa_ref[...], ta_ref[...]], axis=0)
    wa = jnp.where(lab_ref[...] != float(IGNORE), 1.0, 0.0)

    def kd_part(x, nvalid, ncols):
        msk = lax.broadcasted_iota(jnp.int32, (1, ncols), 1) < nvalid
        z = _dg(sat, x, 1, 1) * (1.0 / TAU_C / KD_T)
        z = jnp.where(msk, z, -12000.0)
        zs = z[:BLK]
        zt = z[BLK:]
        ezt = jnp.exp(zt)
        ss = jnp.sum(jnp.exp(zs), axis=1, keepdims=True)
        st = jnp.sum(ezt, axis=1, keepdims=True)
        dot = jnp.sum(ezt * (zt - zs), axis=1, keepdims=True)
        kl = dot / st + jnp.log(ss) - jnp.log(st)
        return jnp.sum(kl * wa)

    vp = kd_part(xps[...], NPIX, NPIX_SC)
    vr = kd_part(xrs[...], NREG, NREG_SC)
    accs[0] = accs[0] + vp
    accs[1] = accs[1] + vr
    accs[2] = accs[2] + jnp.sum(wa)

    @pl.when(i == pl.num_programs(0) - 1)
    def _fin():
        den = jnp.maximum(accs[2], 1.0)
        lp_ref[...] = jnp.broadcast_to(
            accs[0] / den * (KD_T * KD_T) * LW_PIX, (1, 1))
        lr_ref[...] = jnp.broadcast_to(
            accs[1] / den * (KD_T * KD_T) * LW_REG, (1, 1))


BLK = 256


def _loss(sa, ta, xpb, xrb, pidxf, ridxf, pptrf, rptrf, cnt, upd, mf, laba):
    f32 = jnp.float32
    blk = BLK
    return pl.pallas_call(
        _loss_body,
        grid=(MAX_SAMPLES // blk,),
        in_specs=[
            pl.BlockSpec((blk, DIM), lambda i: (i, 0)),
            pl.BlockSpec((blk, DIM), lambda i: (i, 0)),
            pl.BlockSpec((NPIX_SC, DIM), lambda i: (0, 0)),
            pl.BlockSpec((NREG_SC, DIM), lambda i: (0, 0)),
            pl.BlockSpec((NPIX_SC, 1), lambda i: (0, 0)),
            pl.BlockSpec((NREG_SC, 1), lambda i: (0, 0)),
            pl.BlockSpec((NUM_CLASSES, 1), lambda i: (0, 0)),
            pl.BlockSpec((NUM_CLASSES, 1), lambda i: (0, 0)),
            pl.BlockSpec((NUM_CLASSES, 1), lambda i: (0, 0)),
            pl.BlockSpec((KPAD, DIM), lambda i: (0, 0)),
            pl.BlockSpec((NUM_CLASSES, DIM), lambda i: (0, 0)),
            pl.BlockSpec((blk, 1), lambda i: (i, 0)),
        ],
        out_specs=[
            pl.BlockSpec((1, 1), lambda i: (0, 0)),
            pl.BlockSpec((1, 1), lambda i: (0, 0)),
        ],
        out_shape=[
            jax.ShapeDtypeStruct((1, 1), f32),
            jax.ShapeDtypeStruct((1, 1), f32),
        ],
        scratch_shapes=[
            pltpu.VMEM((NPIX_SC, DIM), f32),
            pltpu.VMEM((NREG_SC, DIM), f32),
            pltpu.SMEM((4,), f32),
        ],
    )(sa, ta, xpb, xrb, pidxf, ridxf, pptrf, rptrf, cnt, upd, mf, laba)


def kernel(s_feats, t_feats, logits_S, logits_T, labels, W1, gamma, beta, W2,
           seg_queue, pix_queue, seg_ptr, pix_ptr):
    f32 = jnp.float32
    sp = s_feats.transpose(0, 2, 3, 1).reshape(M, 512)
    tp = t_feats.transpose(0, 2, 3, 1).reshape(M, DIM)

    sa, ta, mean_feat, upd, cnt_col, laba = _prep(
        labels, sp, tp, W1, gamma.reshape(1, DIM), beta.reshape(1, DIM), W2)

    pflat = np.concatenate(
        [(np.arange(NUM_CLASSES, dtype=np.int32)[:, None] * PIXEL_MEM
          + _PIDX[None, :]).reshape(NPIX),
         np.zeros(NPIX_SC - NPIX, np.int32)])
    rflat = np.concatenate(
        [(np.arange(NUM_CLASSES, dtype=np.int32)[:, None] * REGION_MEM
          + _RIDX[None, :]).reshape(NREG),
         np.zeros(NREG_SC - NREG, np.int32)])
    xpb, xrb = _sc_gather(
        pix_queue.reshape(NUM_CLASSES * PIXEL_MEM, DIM), jnp.asarray(pflat),
        seg_queue.reshape(NUM_CLASSES * REGION_MEM, DIM), jnp.asarray(rflat))

    pidxf = jnp.asarray(
        np.concatenate([np.tile(_PIDX, NUM_CLASSES),
                        np.zeros(NPIX_SC - NPIX, np.int32)])
        .astype(np.float32).reshape(NPIX_SC, 1))
    ridxf = jnp.asarray(
        np.concatenate([np.tile(_RIDX, NUM_CLASSES),
                        np.full(NREG_SC - NREG, -1, np.int32)])
        .astype(np.float32).reshape(NREG_SC, 1))
    pptrf = pix_ptr.astype(f32).reshape(NUM_CLASSES, 1)
    rptrf = seg_ptr.astype(f32).reshape(NUM_CLASSES, 1)

    lp_out, lr_out = _loss(sa, ta, xpb, xrb, pidxf, ridxf, pptrf, rptrf,
                           cnt_col, upd, mean_feat, laba)
    return (lp_out[0, 0], lr_out[0, 0])
